# Initial kernel scaffold; baseline (speedup 1.0000x reference)
import jax
import jax.numpy as jnp
from jax.experimental import pallas as pl


def kernel(c_feat, g_feat, edge_index, W, b):
    raise NotImplementedError("write your pallas kernel here")


# speedup vs baseline: 1.1477x; 1.1477x over previous
import functools

import jax
import jax.numpy as jnp
from jax import lax
from jax.experimental import pallas as pl
from jax.experimental.pallas import tpu as pltpu
from jax.experimental.pallas import tpu_sc as plsc

N_NODES = 10000
D = 256
L = 16
NC, NS = 2, 16
NW = NC * NS
DCH = D // L


def _sc_body(ngw, c_hbm, g_hbm, src_hbm, dst_hbm, w_hbm, b_hbm, out_hbm,
             src_v, dst_v, crows, grows, wv, bv, rbuf, ostage, sem_c, sem_g):
    wid = lax.axis_index("s") * NC + lax.axis_index("c")
    ew = ngw * L
    base = wid * ew

    pltpu.sync_copy(src_hbm.at[pl.ds(base, ew)], src_v)
    pltpu.sync_copy(dst_hbm.at[pl.ds(base, ew)], dst_v)
    pltpu.sync_copy(w_hbm, wv)
    pltpu.sync_copy(b_hbm, bv)

    wregs = [wv[pl.ds(j * L, L)] for j in range(DCH)]
    bvec = bv[...]
    lane_iota = lax.iota(jnp.int32, L)

    def lane_xor(v, d, m):
        rbuf[pl.ds(0, L)] = v
        rbuf[pl.ds(L, L)] = v
        if d == L // 2:
            return rbuf[pl.ds(d, L)]
        return jnp.where(m, rbuf[pl.ds(d, L)], rbuf[pl.ds(L - d, L)])

    def group(g, carry):
        cp = pltpu.async_copy(c_hbm.at[src_v.at[pl.ds(g * L, L)]], crows, sem_c)
        gp = pltpu.async_copy(g_hbm.at[dst_v.at[pl.ds(g * L, L)]], grows, sem_g)
        cp.wait()
        gp.wait()
        accs = []
        for i in range(L):
            acc = crows[i, pl.ds(0, L)] * grows[i, pl.ds(0, L)] * wregs[0]
            for j in range(1, DCH):
                acc = acc + (crows[i, pl.ds(j * L, L)]
                             * grows[i, pl.ds(j * L, L)] * wregs[j])
            accs.append(acc)
        d = L // 2
        while len(accs) > 1:
            m = (lane_iota % (2 * d)) < d
            nxt = []
            for i in range(len(accs) // 2):
                a, bq = accs[i], accs[i + len(accs) // 2]
                lo = jnp.where(m, a, bq)
                hi = jnp.where(m, bq, a)
                nxt.append(lo + lane_xor(hi, d, m))
            accs = nxt
            d //= 2
        pre = accs[0] + bvec
        ostage[pl.ds(g * L, L)] = 1.0 / (1.0 + jnp.exp(-pre))
        return carry

    lax.fori_loop(0, ngw, group, 0)
    pltpu.sync_copy(ostage, out_hbm.at[pl.ds(base, ew)])


def kernel(c_feat, g_feat, edge_index, W, b):
    E = edge_index.shape[1]
    epad = -E % (NW * L)
    e_tot = E + epad
    ngw = e_tot // (NW * L)

    src = edge_index[0].astype(jnp.int32)
    dst = edge_index[1].astype(jnp.int32)
    if epad:
        zpad = jnp.zeros((epad,), jnp.int32)
        src = jnp.concatenate([src, zpad])
        dst = jnp.concatenate([dst, zpad])
    w = W[:, 0]
    b16 = jnp.broadcast_to(b, (L,))

    mesh = plsc.VectorSubcoreMesh(core_axis_name="c", subcore_axis_name="s")
    ew = ngw * L
    run = functools.partial(
        pl.kernel,
        out_type=jax.ShapeDtypeStruct((e_tot,), jnp.float32),
        mesh=mesh,
        scratch_types=[
            pltpu.VMEM((ew,), jnp.int32),
            pltpu.VMEM((ew,), jnp.int32),
            pltpu.VMEM((L, D), jnp.float32),
            pltpu.VMEM((L, D), jnp.float32),
            pltpu.VMEM((D,), jnp.float32),
            pltpu.VMEM((L,), jnp.float32),
            pltpu.VMEM((2 * L,), jnp.float32),
            pltpu.VMEM((ew,), jnp.float32),
            pltpu.SemaphoreType.DMA,
            pltpu.SemaphoreType.DMA,
        ],
    )(functools.partial(_sc_body, ngw))
    out = run(c_feat, g_feat, src, dst, w, b16)
    return out[:E, None]


# speedup vs baseline: 1.7072x; 1.4875x over previous
import functools

import jax
import jax.numpy as jnp
from jax import lax
from jax.experimental import pallas as pl
from jax.experimental.pallas import tpu as pltpu
from jax.experimental.pallas import tpu_sc as plsc

D = 256
L = 16
NC, NS = 2, 16
NW = NC * NS
DCH = D // L
NSLOT = 32


def _sc_body(ngw, c_hbm, g_hbm, src_hbm, dst_hbm, w_hbm, b_hbm, out_hbm,
             src_v, dst_v, ac, ag, bc, bg, wv, bv, rbuf, ostage,
             sac, sag, sbc, sbg):
    wid = lax.axis_index("s") * NC + lax.axis_index("c")
    ew = ngw * L
    base = wid * ew

    pltpu.sync_copy(src_hbm.at[pl.ds(base, ew)], src_v)
    pltpu.sync_copy(dst_hbm.at[pl.ds(base, ew)], dst_v)
    pltpu.sync_copy(w_hbm, wv)
    pltpu.sync_copy(b_hbm, bv)

    wregs = [wv[pl.ds(j * L, L)] for j in range(DCH)]
    bvec = bv[...]
    lane_iota = lax.iota(jnp.int32, L)
    masks = {d: (lane_iota % (2 * d)) < d for d in (1, 2, 4, 8)}

    def start(g, rc, rg, sc, sg):
        pltpu.async_copy(c_hbm.at[src_v.at[pl.ds(g * L, L)]], rc, sc)
        pltpu.async_copy(g_hbm.at[dst_v.at[pl.ds(g * L, L)]], rg, sg)

    def wait(rc, rg, sc, sg):
        pltpu.make_async_copy(
            c_hbm.at[src_v.at[pl.ds(0, L)]], rc, sc).wait()
        pltpu.make_async_copy(
            g_hbm.at[dst_v.at[pl.ds(0, L)]], rg, sg).wait()

    def compute(g, rc, rg, slot_base):
        slot = [slot_base]

        def dot(i):
            a0 = rc[i, pl.ds(0, L)] * rg[i, pl.ds(0, L)] * wregs[0]
            a1 = (rc[i, pl.ds(8 * L, L)] * rg[i, pl.ds(8 * L, L)]
                  * wregs[8])
            for j in range(1, 8):
                a0 = a0 + (rc[i, pl.ds(j * L, L)]
                           * rg[i, pl.ds(j * L, L)] * wregs[j])
                a1 = a1 + (rc[i, pl.ds((j + 8) * L, L)]
                           * rg[i, pl.ds((j + 8) * L, L)] * wregs[j + 8])
            return a0 + a1

        def lane_xor(v, d):
            off = (slot[0] % NSLOT) * (2 * L)
            slot[0] += 1
            rbuf[pl.ds(off, L)] = v
            rbuf[pl.ds(off + L, L)] = v
            if d == L // 2:
                return rbuf[pl.ds(off + d, L)]
            return jnp.where(masks[d], rbuf[pl.ds(off + d, L)],
                             rbuf[pl.ds(off + L - d, L)])

        def build(i, n):
            if n == L:
                return dot(i)
            a = build(i, 2 * n)
            b = build(i + n, 2 * n)
            m = masks[n]
            lo = jnp.where(m, a, b)
            hi = jnp.where(m, b, a)
            return lo + lane_xor(hi, n)

        pre = build(0, 1) + bvec
        ostage[pl.ds(g * L, L)] = 1.0 / (1.0 + jnp.exp(-pre))

    niter = ngw // 2
    start(0, ac, ag, sac, sag)

    def body(t, carry):
        g0 = 2 * t
        start(g0 + 1, bc, bg, sbc, sbg)
        wait(ac, ag, sac, sag)
        compute(g0, ac, ag, 0)

        @pl.when(t < niter - 1)
        def _():
            start(g0 + 2, ac, ag, sac, sag)

        wait(bc, bg, sbc, sbg)
        compute(g0 + 1, bc, bg, 16)
        return carry

    lax.fori_loop(0, niter, body, 0)
    pltpu.sync_copy(ostage, out_hbm.at[pl.ds(base, ew)])


def kernel(c_feat, g_feat, edge_index, W, b):
    E = edge_index.shape[1]
    epad = -E % (2 * NW * L)
    e_tot = E + epad
    ngw = e_tot // (NW * L)

    src = edge_index[0].astype(jnp.int32)
    dst = edge_index[1].astype(jnp.int32)
    if epad:
        zpad = jnp.zeros((epad,), jnp.int32)
        src = jnp.concatenate([src, zpad])
        dst = jnp.concatenate([dst, zpad])
    w = W[:, 0]
    b16 = jnp.broadcast_to(b, (L,))

    mesh = plsc.VectorSubcoreMesh(core_axis_name="c", subcore_axis_name="s")
    ew = ngw * L
    run = functools.partial(
        pl.kernel,
        out_type=jax.ShapeDtypeStruct((e_tot,), jnp.float32),
        mesh=mesh,
        scratch_types=[
            pltpu.VMEM((ew,), jnp.int32),
            pltpu.VMEM((ew,), jnp.int32),
            pltpu.VMEM((L, D), jnp.float32),
            pltpu.VMEM((L, D), jnp.float32),
            pltpu.VMEM((L, D), jnp.float32),
            pltpu.VMEM((L, D), jnp.float32),
            pltpu.VMEM((D,), jnp.float32),
            pltpu.VMEM((L,), jnp.float32),
            pltpu.VMEM((NSLOT * 2 * L,), jnp.float32),
            pltpu.VMEM((ew,), jnp.float32),
            pltpu.SemaphoreType.DMA,
            pltpu.SemaphoreType.DMA,
            pltpu.SemaphoreType.DMA,
            pltpu.SemaphoreType.DMA,
        ],
    )(functools.partial(_sc_body, ngw))
    out = run(c_feat, g_feat, src, dst, w, b16)
    return out[:E, None]


# speedup vs baseline: 1.7305x; 1.0136x over previous
import functools

import jax
import jax.numpy as jnp
from jax import lax
from jax.experimental import pallas as pl
from jax.experimental.pallas import tpu as pltpu
from jax.experimental.pallas import tpu_sc as plsc

D = 256
L = 16
NC, NS = 2, 16
NW = NC * NS
DCH = D // L
GPC = 2
CH = GPC * L
NSLOT = 64


def _sc_body(nch, c_hbm, g_hbm, src_hbm, dst_hbm, w_hbm, b_hbm, out_hbm,
             src_v, dst_v, ac, ag, bc, bg, wv, bv, rbuf, ostage,
             sac, sag, sbc, sbg):
    wid = lax.axis_index("s") * NC + lax.axis_index("c")
    ew = nch * CH
    base = wid * ew

    pltpu.sync_copy(src_hbm.at[pl.ds(base, ew)], src_v)
    pltpu.sync_copy(dst_hbm.at[pl.ds(base, ew)], dst_v)
    pltpu.sync_copy(w_hbm, wv)
    pltpu.sync_copy(b_hbm, bv)

    wregs = [wv[pl.ds(j * L, L)] for j in range(DCH)]
    bvec = bv[...]
    lane_iota = lax.iota(jnp.int32, L)
    masks = {d: (lane_iota % (2 * d)) < d for d in (1, 2, 4, 8)}

    def start(ch, rc, rg, sc, sg):
        pltpu.async_copy(c_hbm.at[src_v.at[pl.ds(ch * CH, CH)]], rc, sc)
        pltpu.async_copy(g_hbm.at[dst_v.at[pl.ds(ch * CH, CH)]], rg, sg)

    def wait(rc, rg, sc, sg):
        pltpu.make_async_copy(
            c_hbm.at[src_v.at[pl.ds(0, CH)]], rc, sc).wait()
        pltpu.make_async_copy(
            g_hbm.at[dst_v.at[pl.ds(0, CH)]], rg, sg).wait()

    def compute(ch, rc, rg, slot_base):
        slot = [slot_base]

        def dot(r):
            a0 = rc[r, pl.ds(0, L)] * rg[r, pl.ds(0, L)] * wregs[0]
            a1 = (rc[r, pl.ds(8 * L, L)] * rg[r, pl.ds(8 * L, L)]
                  * wregs[8])
            for j in range(1, 8):
                a0 = a0 + (rc[r, pl.ds(j * L, L)]
                           * rg[r, pl.ds(j * L, L)] * wregs[j])
                a1 = a1 + (rc[r, pl.ds((j + 8) * L, L)]
                           * rg[r, pl.ds((j + 8) * L, L)] * wregs[j + 8])
            return a0 + a1

        def lane_xor(v, d):
            off = (slot[0] % NSLOT) * (2 * L)
            slot[0] += 1
            rbuf[pl.ds(off, L)] = v
            rbuf[pl.ds(off + L, L)] = v
            if d == L // 2:
                return rbuf[pl.ds(off + d, L)]
            return jnp.where(masks[d], rbuf[pl.ds(off + d, L)],
                             rbuf[pl.ds(off + L - d, L)])

        def build(gg, i, n):
            if n == L:
                return dot(gg * L + i)
            a = build(gg, i, 2 * n)
            b = build(gg, i + n, 2 * n)
            m = masks[n]
            lo = jnp.where(m, a, b)
            hi = jnp.where(m, b, a)
            return lo + lane_xor(hi, n)

        for gg in range(GPC):
            pre = build(gg, 0, 1) + bvec
            ostage[pl.ds((ch * GPC + gg) * L, L)] = (
                1.0 / (1.0 + jnp.exp(-pre)))

    niter = nch // 2
    start(0, ac, ag, sac, sag)

    def body(t, carry):
        ch0 = 2 * t
        start(ch0 + 1, bc, bg, sbc, sbg)
        wait(ac, ag, sac, sag)
        compute(ch0, ac, ag, 0)

        @pl.when(t < niter - 1)
        def _():
            start(ch0 + 2, ac, ag, sac, sag)

        wait(bc, bg, sbc, sbg)
        compute(ch0 + 1, bc, bg, NSLOT // 2)
        return carry

    lax.fori_loop(0, niter, body, 0)
    pltpu.sync_copy(ostage, out_hbm.at[pl.ds(base, ew)])


def kernel(c_feat, g_feat, edge_index, W, b):
    E = edge_index.shape[1]
    epad = -E % (2 * NW * CH)
    e_tot = E + epad
    nch = e_tot // (NW * CH)

    src = edge_index[0].astype(jnp.int32)
    dst = edge_index[1].astype(jnp.int32)
    if epad:
        zpad = jnp.zeros((epad,), jnp.int32)
        src = jnp.concatenate([src, zpad])
        dst = jnp.concatenate([dst, zpad])
    w = W[:, 0]
    b16 = jnp.broadcast_to(b, (L,))

    mesh = plsc.VectorSubcoreMesh(core_axis_name="c", subcore_axis_name="s")
    ew = nch * CH
    run = functools.partial(
        pl.kernel,
        out_type=jax.ShapeDtypeStruct((e_tot,), jnp.float32),
        mesh=mesh,
        scratch_types=[
            pltpu.VMEM((ew,), jnp.int32),
            pltpu.VMEM((ew,), jnp.int32),
            pltpu.VMEM((CH, D), jnp.float32),
            pltpu.VMEM((CH, D), jnp.float32),
            pltpu.VMEM((CH, D), jnp.float32),
            pltpu.VMEM((CH, D), jnp.float32),
            pltpu.VMEM((D,), jnp.float32),
            pltpu.VMEM((L,), jnp.float32),
            pltpu.VMEM((NSLOT * 2 * L,), jnp.float32),
            pltpu.VMEM((ew,), jnp.float32),
            pltpu.SemaphoreType.DMA,
            pltpu.SemaphoreType.DMA,
            pltpu.SemaphoreType.DMA,
            pltpu.SemaphoreType.DMA,
        ],
    )(functools.partial(_sc_body, nch))
    out = run(c_feat, g_feat, src, dst, w, b16)
    return out[:E, None]
